# baseline (speedup 1.0000x reference)
import functools

import jax
import jax.numpy as jnp
from jax import lax
from jax.experimental import pallas as pl
from jax.experimental.pallas import tpu as pltpu
from jax.experimental.pallas import tpu_sc as plsc

_CHUNK = 128


@functools.lru_cache(maxsize=None)
def _make_lookup(vocab: int, embed: int, batch: int):
  info = plsc.get_sparse_core_info()
  nc, ns = info.num_cores, info.num_subcores
  nw = nc * ns
  assert batch % (nw * _CHUNK) == 0
  ch = batch // (nw * _CHUNK)
  mesh = plsc.VectorSubcoreMesh(core_axis_name="c", subcore_axis_name="s")

  @functools.partial(
      pl.kernel,
      mesh=mesh,
      out_type=jax.ShapeDtypeStruct((nw * ch, _CHUNK, embed), jnp.float32),
      scratch_types=[
          pltpu.VMEM((ch, _CHUNK), jnp.int32),
          pltpu.VMEM((ch, _CHUNK, embed), jnp.float32),
          pltpu.SemaphoreType.DMA,
      ],
      compiler_params=pltpu.CompilerParams(use_tc_tiling_on_sc=False),
  )
  def lookup(emb_hbm, idx_hbm, out_hbm, idx_v, rows_v, sem):
    wid = lax.axis_index("s") * nc + lax.axis_index("c")
    base = wid * ch
    pltpu.sync_copy(idx_hbm.at[pl.ds(base, ch)], idx_v)
    copies = [
        pltpu.async_copy(emb_hbm.at[idx_v.at[j]], rows_v.at[j], sem)
        for j in range(ch)
    ]
    for cp in copies:
      cp.wait()
    pltpu.sync_copy(rows_v, out_hbm.at[pl.ds(base, ch)])

  return lookup


def kernel(inputs, embeddings):
  batch, = inputs.shape
  vocab, embed = embeddings.shape
  idx = inputs.astype(jnp.int32).reshape(batch // _CHUNK, _CHUNK)
  out = _make_lookup(vocab, embed, batch)(embeddings, idx)
  return out.reshape(batch, embed)


# speedup vs baseline: 3.8269x; 3.8269x over previous
import functools

import jax
import jax.numpy as jnp
from jax import lax
from jax.experimental import pallas as pl
from jax.experimental.pallas import tpu as pltpu
from jax.experimental.pallas import tpu_sc as plsc

_NBUF = 16
_L = 16


@functools.lru_cache(maxsize=None)
def _make_lookup(vocab: int, embed: int, batch: int):
  info = plsc.get_sparse_core_info()
  nc, ns = info.num_cores, info.num_subcores
  nw = nc * ns
  assert batch % nw == 0
  bpw = batch // nw
  assert bpw % _NBUF == 0
  mesh = plsc.VectorSubcoreMesh(core_axis_name="c", subcore_axis_name="s")

  @functools.partial(
      pl.kernel,
      mesh=mesh,
      out_type=jax.ShapeDtypeStruct((embed, batch), jnp.float32),
      scratch_types=[
          pltpu.VMEM((bpw,), jnp.int32),
          pltpu.VMEM((bpw * embed,), jnp.float32),
          pltpu.VMEM((embed, bpw), jnp.float32),
          pltpu.SemaphoreType.DMA,
      ]
      + [pltpu.VMEM((embed, 128), jnp.float32) for _ in range(_NBUF)]
      + [pltpu.SemaphoreType.DMA for _ in range(_NBUF)],
      compiler_params=pltpu.CompilerParams(needs_layout_passes=False),
  )
  def lookup(emb_hbm, idx_hbm, out_hbm, idx_s, rows_v, t_v, sem_i, *ring):
    slabs = ring[:_NBUF]
    sems = ring[_NBUF:]
    iota = lax.iota(jnp.int32, _L)
    wid = lax.axis_index("s") * nc + lax.axis_index("c")
    base = wid * bpw
    pltpu.async_copy(idx_hbm.at[pl.ds(base, bpw)], idx_s, sem_i).wait()

    def fire(r, s):
      cb = pl.multiple_of((r // 128) * 128, 128)
      return pltpu.async_copy(
          emb_hbm.at[:, pl.ds(cb, 128)], slabs[s], sems[s]
      )

    def extract(r, i, s):
      lane = jnp.full((_L,), r % 128, jnp.int32)
      for h in range(embed // _L):
        col = plsc.load_gather(slabs[s], [h * _L + iota, lane])
        rows_v[pl.ds(i * embed + h * _L, _L)] = col

    rv0 = idx_s[pl.ds(0, _NBUF)]
    for s in range(_NBUF):
      fire(rv0[s], s)

    def step(g, carry):
      i = g * _NBUF
      rv = idx_s[pl.ds(i, _NBUF)]
      for s in range(_NBUF):
        pltpu.make_async_copy(
            emb_hbm.at[:, pl.ds(0, 128)], slabs[s], sems[s]
        ).wait()
        extract(rv[s], i + s, s)

        @pl.when(g + 1 < bpw // _NBUF)
        def _():
          rvn = idx_s[pl.ds(i + _NBUF, _NBUF)]
          fire(rvn[s], s)

      return carry

    lax.fori_loop(0, bpw // _NBUF, step, 0)

    def tstep(g, carry):
      b16 = (g * _L + iota) * embed
      for j in range(embed):
        t_v[j, pl.ds(g * _L, _L)] = plsc.load_gather(rows_v, [b16 + j])
      return carry

    lax.fori_loop(0, bpw // _L, tstep, 0)
    pltpu.sync_copy(t_v, out_hbm.at[:, pl.ds(base, bpw)])

  return lookup


def kernel(inputs, embeddings):
  batch, = inputs.shape
  vocab, embed = embeddings.shape
  idx = inputs.astype(jnp.int32)
  out_t = _make_lookup(vocab, embed, batch)(embeddings.T, idx)
  return out_t.T
